# baseline (speedup 1.0000x reference)
import functools

import jax
import jax.numpy as jnp
from jax import lax
from jax.experimental import pallas as pl
from jax.experimental.pallas import tpu as pltpu
from jax.experimental.pallas import tpu_sc as plsc

N_IT = 1000
N_PE = 1000
N_GR = 1000
W = 1024
BIG = 1000.0
_HL2PI = 0.9189385332046727

NW = 32
CHUNK = 2000
B_TOT = 1000000
NCHUNK = B_TOT // CHUNK
CH_LO = NCHUNK // NW
RICH = NCHUNK - NW * CH_LO


def _prep_body(a_ref, bb_ref, cpad_ref, t_ref, f_ref, asp_ref, cvec_ref):
    col = lax.broadcasted_iota(jnp.int32, (N_IT, W), 1)
    bb = jnp.broadcast_to(bb_ref[...], (N_IT, W))
    c = jnp.where(col == 0, bb,
                  jnp.where(col <= N_GR - 2, jax.nn.softplus(cpad_ref[...]), 0.0))
    ri = lax.broadcasted_iota(jnp.int32, (W, W), 0)
    ci = lax.broadcasted_iota(jnp.int32, (W, W), 1)
    ustrict = (ri < ci).astype(jnp.float32)
    bshift = lax.dot_general(c, ustrict, (((1,), (0,)), ((), ())),
                             preferred_element_type=jnp.float32)
    bmat = bshift + c
    f_ref[...] = jnp.where(col == 0, -BIG, jnp.where(col >= N_GR, BIG, bshift))
    a = jax.nn.softplus(a_ref[...])
    asp_ref[...] = a
    t = t_ref[...]
    sb = jnp.sum(jnp.where(col <= N_GR - 2, bmat * bmat, 0.0))
    s_all = jnp.sum(a * a) + sb + jnp.sum(t * t)
    n_terms = N_IT + N_IT * (N_GR - 1) + N_PE
    log_prior = -0.5 * s_all - n_terms * _HL2PI
    cvec_ref[...] = jnp.full((16,), -1e6 * log_prior, dtype=jnp.float32)


def _prep(a_, b_base_, b_diff_pad, t):
    return pl.pallas_call(
        _prep_body,
        out_shape=(
            jax.ShapeDtypeStruct((N_IT, W), jnp.float32),
            jax.ShapeDtypeStruct((N_IT,), jnp.float32),
            jax.ShapeDtypeStruct((16,), jnp.float32),
        ),
    )(a_, b_base_, b_diff_pad, t)


def _log_poly(p):
    bits = lax.bitcast_convert_type(p, jnp.int32)
    e = lax.shift_right_logical(bits, 23) - 127
    mbits = jnp.bitwise_or(jnp.bitwise_and(bits, 0x007FFFFF), 0x3F800000)
    m = lax.bitcast_convert_type(mbits, jnp.float32)
    big_m = m >= 1.4142135623730951
    m = jnp.where(big_m, m * 0.5, m)
    e = e + jnp.where(big_m, 1, 0)
    z = (m - 1.0) / (m + 1.0)
    z2 = z * z
    s = 2.0 * z * (1.0 + z2 * (0.3333333333 + z2 * (0.2 + z2 * 0.14285715)))
    return e.astype(jnp.float32) * 0.6931471805599453 + s


def _sc_body(f0_hbm, pe_hbm, tbl_hbm, a_hbm, t_hbm, c_hbm, out_hbm,
             pe_v, f0_v, f1_v, bl_v, bu_v, aa_v, tt_v, out_v,
             a_v, t_v, c_v, tbl_s, sem):
    nc = 2
    sid = lax.axis_index("s")
    w = sid * nc + lax.axis_index("c")
    @pl.when(sid == 0)
    def _stage_table():
        pltpu.sync_copy(tbl_hbm, tbl_s)
    pltpu.sync_copy(a_hbm, a_v)
    pltpu.sync_copy(t_hbm, t_v)
    pltpu.sync_copy(c_hbm, c_v)
    plsc.subcore_barrier()
    cvec = c_v[...]
    lanes = lax.iota(jnp.int32, 16)
    zeros = jnp.zeros((16,), jnp.int32)
    ones = jnp.full((16,), 1, jnp.int32)
    twos = jnp.full((16,), 2, jnp.int32)

    start_ch = CH_LO * w + jnp.minimum(w, RICH)
    n_ch = CH_LO + jnp.where(w < RICH, 1, 0)

    def fire(j, b):
        base = (start_ch + j) * CHUNK
        pltpu.sync_copy(f0_hbm.at[pl.ds(base, CHUNK)], f0_v[b])
        pltpu.sync_copy(pe_hbm.at[pl.ds(base, CHUNK)], pe_v)
        @plsc.parallel_loop(0, CHUNK, step=16, unroll=8)
        def _pass_a(e):
            f = f0_v[b][pl.ds(e, 16)]
            pers = pe_v[pl.ds(e, 16)]
            f1_v[b][pl.ds(e, 16)] = f + 1
            item = lax.shift_right_logical(f, 10)
            aa_v[b][pl.ds(e, 16)] = plsc.load_gather(a_v, [item])
            tt_v[b][pl.ds(e, 16)] = plsc.load_gather(t_v, [pers])
        pltpu.async_copy(tbl_s.at[f0_v[b]], bl_v[b], sem[b])
        pltpu.async_copy(tbl_s.at[f1_v[b]], bu_v[b], sem[b])

    def drain(j, b):
        base = (start_ch + j) * CHUNK
        pltpu.make_async_copy(tbl_s.at[f0_v[b]], bl_v[b], sem[b]).wait()
        pltpu.make_async_copy(tbl_s.at[f1_v[b]], bu_v[b], sem[b]).wait()
        @plsc.parallel_loop(0, CHUNK, step=16, unroll=8)
        def _pass_b(e):
            bl = bl_v[b][pl.ds(e, 16)]
            bu = bu_v[b][pl.ds(e, 16)]
            aa = aa_v[b][pl.ds(e, 16)]
            tt = tt_v[b][pl.ds(e, 16)]
            x1 = jnp.clip(aa * (tt - bl), -85.0, 85.0)
            x2 = jnp.clip(aa * (tt - bu), -85.0, 85.0)
            s1 = 1.0 / (1.0 + jnp.exp(-x1))
            s2 = 1.0 / (1.0 + jnp.exp(-x2))
            out_v[pl.ds(e, 16)] = cvec - _log_poly(s1 - s2)
        pltpu.sync_copy(out_v, out_hbm.at[pl.ds(base, CHUNK)])

    fire(0, 0)

    def pair_body(jj, carry):
        for par in range(2):
            j = 2 * jj + par
            jn = j + 1

            @pl.when(jn < n_ch)
            def _f():
                fire(jn, (par + 1) % 2)

            @pl.when(j < n_ch)
            def _d():
                drain(j, par)
        return carry

    lax.fori_loop(0, (CH_LO + 2) // 2, pair_body, 0)


@functools.partial(jax.jit, static_argnums=())
def _sc_call(f0, pe, tbl, asp, t, cvec):
    mesh = plsc.VectorSubcoreMesh(core_axis_name="c", subcore_axis_name="s")
    return pl.kernel(
        _sc_body,
        out_type=jax.ShapeDtypeStruct((B_TOT,), jnp.float32),
        mesh=mesh,
        scratch_types=[
            pltpu.VMEM((CHUNK,), jnp.int32),
            [pltpu.VMEM((CHUNK,), jnp.int32)] * 2,
            [pltpu.VMEM((CHUNK,), jnp.int32)] * 2,
            [pltpu.VMEM((CHUNK,), jnp.float32)] * 2,
            [pltpu.VMEM((CHUNK,), jnp.float32)] * 2,
            [pltpu.VMEM((CHUNK,), jnp.float32)] * 2,
            [pltpu.VMEM((CHUNK,), jnp.float32)] * 2,
            pltpu.VMEM((CHUNK,), jnp.float32),
            pltpu.VMEM((N_IT,), jnp.float32),
            pltpu.VMEM((N_PE,), jnp.float32),
            pltpu.VMEM((16,), jnp.float32),
            pltpu.VMEM_SHARED((N_IT * W,), jnp.float32),
            [pltpu.SemaphoreType.DMA] * 2,
        ],
        compiler_params=pltpu.CompilerParams(needs_layout_passes=False),
    )(f0, pe, tbl, asp, t, cvec)


def kernel(indices, a_, b_base_, b_diff_, t):
    b = indices.shape[0]
    b_diff_pad = jnp.pad(b_diff_, ((0, 0), (1, W - 1 - b_diff_.shape[1])))
    fmat, asp, cvec = _prep(a_, b_base_, b_diff_pad, t)
    tbl = fmat.reshape(N_IT * W)
    del b
    wf = jnp.array([W, 0, 1], dtype=jnp.int32)
    wp = jnp.array([0, 1, 0], dtype=jnp.int32)
    f0 = jnp.sum(indices * wf, axis=1) - 1
    pe = jnp.sum(indices * wp, axis=1)
    return _sc_call(f0, pe, tbl, asp, t, cvec)


# speedup vs baseline: 1.2443x; 1.2443x over previous
import functools

import jax
import jax.numpy as jnp
from jax import lax
from jax.experimental import pallas as pl
from jax.experimental.pallas import tpu as pltpu
from jax.experimental.pallas import tpu_sc as plsc

N_IT = 1000
N_PE = 1000
N_GR = 1000
W = 1024
BIG = 1000.0
_HL2PI = 0.9189385332046727

NW = 32
CHUNK = 2000
B_TOT = 1000000
NCHUNK = B_TOT // CHUNK
CH_LO = NCHUNK // NW
RICH = NCHUNK - NW * CH_LO


def _prep_body(a_ref, bb_ref, cpad_ref, t_ref, f_ref, asp_ref, cvec_ref):
    col = lax.broadcasted_iota(jnp.int32, (N_IT, W), 1)
    bb = jnp.broadcast_to(bb_ref[...], (N_IT, W))
    c = jnp.where(col == 0, bb,
                  jnp.where(col <= N_GR - 2, jax.nn.softplus(cpad_ref[...]), 0.0))
    ri = lax.broadcasted_iota(jnp.int32, (W, W), 0)
    ci = lax.broadcasted_iota(jnp.int32, (W, W), 1)
    ustrict = (ri < ci).astype(jnp.float32)
    bshift = lax.dot_general(c, ustrict, (((1,), (0,)), ((), ())),
                             preferred_element_type=jnp.float32)
    bmat = bshift + c
    f_ref[...] = jnp.where(col == 0, -BIG, jnp.where(col >= N_GR, BIG, bshift))
    a = jax.nn.softplus(a_ref[...])
    asp_ref[...] = a
    t = t_ref[...]
    sb = jnp.sum(jnp.where(col <= N_GR - 2, bmat * bmat, 0.0))
    s_all = jnp.sum(a * a) + sb + jnp.sum(t * t)
    n_terms = N_IT + N_IT * (N_GR - 1) + N_PE
    log_prior = -0.5 * s_all - n_terms * _HL2PI
    cvec_ref[...] = jnp.full((16,), -1e6 * log_prior, dtype=jnp.float32)


def _prep(a_, b_base_, b_diff_pad, t):
    return pl.pallas_call(
        _prep_body,
        out_shape=(
            jax.ShapeDtypeStruct((N_IT, W), jnp.float32),
            jax.ShapeDtypeStruct((N_IT,), jnp.float32),
            jax.ShapeDtypeStruct((16,), jnp.float32),
        ),
    )(a_, b_base_, b_diff_pad, t)


def _log_poly(p):
    bits = lax.bitcast_convert_type(p, jnp.int32)
    e = lax.shift_right_logical(bits, 23) - 127
    mbits = jnp.bitwise_or(jnp.bitwise_and(bits, 0x007FFFFF), 0x3F800000)
    m = lax.bitcast_convert_type(mbits, jnp.float32)
    big_m = m >= 1.4142135623730951
    m = jnp.where(big_m, m * 0.5, m)
    e = e + jnp.where(big_m, 1, 0)
    z = (m - 1.0) / (m + 1.0)
    z2 = z * z
    s = 2.0 * z * (1.0 + z2 * (0.3333333333 + z2 * (0.2 + z2 * 0.14285715)))
    return e.astype(jnp.float32) * 0.6931471805599453 + s


def _sc_body(pk_hbm, tbl_hbm, a_hbm, t_hbm, c_hbm, out_hbm,
             pk_v, f0_v, f1_v, bl_v, bu_v, aa_v, tt_v, out_v,
             a_v, t_v, c_v, tbl_s, sem):
    nc = 2
    sid = lax.axis_index("s")
    w = sid * nc + lax.axis_index("c")
    @pl.when(sid == 0)
    def _stage_table():
        pltpu.sync_copy(tbl_hbm, tbl_s)
    pltpu.sync_copy(a_hbm, a_v)
    pltpu.sync_copy(t_hbm, t_v)
    pltpu.sync_copy(c_hbm, c_v)
    plsc.subcore_barrier()
    cvec = c_v[...]
    lanes = lax.iota(jnp.int32, 16)
    zeros = jnp.zeros((16,), jnp.int32)
    ones = jnp.full((16,), 1, jnp.int32)
    twos = jnp.full((16,), 2, jnp.int32)

    start_ch = CH_LO * w + jnp.minimum(w, RICH)
    n_ch = CH_LO + jnp.where(w < RICH, 1, 0)

    def fire(j, b):
        base = (start_ch + j) * CHUNK
        pltpu.sync_copy(pk_hbm.at[pl.ds(base, CHUNK)], pk_v)
        @plsc.parallel_loop(0, CHUNK, step=16, unroll=8)
        def _pass_a(e):
            pk = pk_v[pl.ds(e, 16)]
            pers = jnp.bitwise_and(pk, 1023)
            f = lax.shift_right_logical(pk, 10) - 1
            f0_v[b][pl.ds(e, 16)] = f
            f1_v[b][pl.ds(e, 16)] = f + 1
            item = lax.shift_right_logical(pk, 20)
            aa_v[b][pl.ds(e, 16)] = plsc.load_gather(a_v, [item])
            tt_v[b][pl.ds(e, 16)] = plsc.load_gather(t_v, [pers])
        pltpu.async_copy(tbl_s.at[f0_v[b]], bl_v[b], sem[b])
        pltpu.async_copy(tbl_s.at[f1_v[b]], bu_v[b], sem[b])

    def drain(j, b):
        base = (start_ch + j) * CHUNK
        pltpu.make_async_copy(tbl_s.at[f0_v[b]], bl_v[b], sem[b]).wait()
        pltpu.make_async_copy(tbl_s.at[f1_v[b]], bu_v[b], sem[b]).wait()
        @plsc.parallel_loop(0, CHUNK, step=16, unroll=8)
        def _pass_b(e):
            bl = bl_v[b][pl.ds(e, 16)]
            bu = bu_v[b][pl.ds(e, 16)]
            aa = aa_v[b][pl.ds(e, 16)]
            tt = tt_v[b][pl.ds(e, 16)]
            x1 = jnp.clip(aa * (tt - bl), -85.0, 85.0)
            x2 = jnp.clip(aa * (tt - bu), -85.0, 85.0)
            s1 = 1.0 / (1.0 + jnp.exp(-x1))
            s2 = 1.0 / (1.0 + jnp.exp(-x2))
            out_v[pl.ds(e, 16)] = cvec - _log_poly(s1 - s2)
        pltpu.sync_copy(out_v, out_hbm.at[pl.ds(base, CHUNK)])

    fire(0, 0)

    def pair_body(jj, carry):
        for par in range(2):
            j = 2 * jj + par
            jn = j + 1

            @pl.when(jn < n_ch)
            def _f():
                fire(jn, (par + 1) % 2)

            @pl.when(j < n_ch)
            def _d():
                drain(j, par)
        return carry

    lax.fori_loop(0, (CH_LO + 2) // 2, pair_body, 0)


@functools.partial(jax.jit, static_argnums=())
def _sc_call(pk, tbl, asp, t, cvec):
    mesh = plsc.VectorSubcoreMesh(core_axis_name="c", subcore_axis_name="s")
    return pl.kernel(
        _sc_body,
        out_type=jax.ShapeDtypeStruct((B_TOT,), jnp.float32),
        mesh=mesh,
        scratch_types=[
            pltpu.VMEM((CHUNK,), jnp.int32),
            [pltpu.VMEM((CHUNK,), jnp.int32)] * 2,
            [pltpu.VMEM((CHUNK,), jnp.int32)] * 2,
            [pltpu.VMEM((CHUNK,), jnp.float32)] * 2,
            [pltpu.VMEM((CHUNK,), jnp.float32)] * 2,
            [pltpu.VMEM((CHUNK,), jnp.float32)] * 2,
            [pltpu.VMEM((CHUNK,), jnp.float32)] * 2,
            pltpu.VMEM((CHUNK,), jnp.float32),
            pltpu.VMEM((N_IT,), jnp.float32),
            pltpu.VMEM((N_PE,), jnp.float32),
            pltpu.VMEM((16,), jnp.float32),
            pltpu.VMEM_SHARED((N_IT * W,), jnp.float32),
            [pltpu.SemaphoreType.DMA] * 2,
        ],
        compiler_params=pltpu.CompilerParams(needs_layout_passes=False),
    )(pk, tbl, asp, t, cvec)


def kernel(indices, a_, b_base_, b_diff_, t):
    b = indices.shape[0]
    b_diff_pad = jnp.pad(b_diff_, ((0, 0), (1, W - 1 - b_diff_.shape[1])))
    fmat, asp, cvec = _prep(a_, b_base_, b_diff_pad, t)
    tbl = fmat.reshape(N_IT * W)
    del b
    wpk = jnp.array([1 << 20, 1, 1 << 10], dtype=jnp.int32)
    pk = jnp.sum(indices * wpk, axis=1)
    return _sc_call(pk, tbl, asp, t, cvec)


# speedup vs baseline: 1.2459x; 1.0013x over previous
import functools

import jax
import jax.numpy as jnp
from jax import lax
from jax.experimental import pallas as pl
from jax.experimental.pallas import tpu as pltpu
from jax.experimental.pallas import tpu_sc as plsc

N_IT = 1000
N_PE = 1000
N_GR = 1000
W = 1024
BIG = 1000.0
_HL2PI = 0.9189385332046727

NW = 32
CHUNK = 2000
B_TOT = 1000000
NCHUNK = B_TOT // CHUNK
CH_LO = NCHUNK // NW
RICH = NCHUNK - NW * CH_LO


def _prep_body(a_ref, bb_ref, cpad_ref, t_ref, f_ref, asp_ref, cvec_ref):
    col = lax.broadcasted_iota(jnp.int32, (N_IT, W), 1)
    bb = jnp.broadcast_to(bb_ref[...], (N_IT, W))
    c = jnp.where(col == 0, bb,
                  jnp.where(col <= N_GR - 2, jax.nn.softplus(cpad_ref[...]), 0.0))
    ri = lax.broadcasted_iota(jnp.int32, (W, W), 0)
    ci = lax.broadcasted_iota(jnp.int32, (W, W), 1)
    ustrict = (ri < ci).astype(jnp.float32)
    bshift = lax.dot_general(c, ustrict, (((1,), (0,)), ((), ())),
                             preferred_element_type=jnp.float32)
    bmat = bshift + c
    f_ref[...] = jnp.where(col == 0, -BIG, jnp.where(col >= N_GR, BIG, bshift))
    a = jax.nn.softplus(a_ref[...])
    asp_ref[...] = a
    t = t_ref[...]
    sb = jnp.sum(jnp.where(col <= N_GR - 2, bmat * bmat, 0.0))
    s_all = jnp.sum(a * a) + sb + jnp.sum(t * t)
    n_terms = N_IT + N_IT * (N_GR - 1) + N_PE
    log_prior = -0.5 * s_all - n_terms * _HL2PI
    cvec_ref[...] = jnp.full((16,), -1e6 * log_prior, dtype=jnp.float32)


def _prep(a_, b_base_, b_diff_pad, t):
    return pl.pallas_call(
        _prep_body,
        out_shape=(
            jax.ShapeDtypeStruct((N_IT, W), jnp.float32),
            jax.ShapeDtypeStruct((N_IT,), jnp.float32),
            jax.ShapeDtypeStruct((16,), jnp.float32),
        ),
    )(a_, b_base_, b_diff_pad, t)


def _log_poly(p):
    bits = lax.bitcast_convert_type(p, jnp.int32)
    e = lax.shift_right_logical(bits, 23) - 127
    mbits = jnp.bitwise_or(jnp.bitwise_and(bits, 0x007FFFFF), 0x3F800000)
    m = lax.bitcast_convert_type(mbits, jnp.float32)
    big_m = m >= 1.4142135623730951
    m = jnp.where(big_m, m * 0.5, m)
    e = e + jnp.where(big_m, 1, 0)
    z = (m - 1.0) / (m + 1.0)
    z2 = z * z
    s = 2.0 * z * (1.0 + z2 * (0.3333333333 + z2 * (0.2 + z2 * 0.14285715)))
    return e.astype(jnp.float32) * 0.6931471805599453 + s


def _sc_body(pk_hbm, tbl_hbm, a_hbm, t_hbm, c_hbm, out_hbm,
             pk_v, f0_v, f1_v, bl_v, bu_v, aa_v, tt_v, out_v,
             a_v, t_v, c_v, tbl_s, sem):
    nc = 2
    sid = lax.axis_index("s")
    w = sid * nc + lax.axis_index("c")
    @pl.when(sid == 0)
    def _stage_table():
        pltpu.sync_copy(tbl_hbm, tbl_s)
    pltpu.sync_copy(a_hbm, a_v)
    pltpu.sync_copy(t_hbm, t_v)
    pltpu.sync_copy(c_hbm, c_v)
    plsc.subcore_barrier()
    cvec = c_v[...]
    start_ch = CH_LO * w + jnp.minimum(w, RICH)
    n_ch = CH_LO + jnp.where(w < RICH, 1, 0)

    def fire(j, b):
        base = (start_ch + j) * CHUNK
        pltpu.sync_copy(pk_hbm.at[pl.ds(base, CHUNK)], pk_v)
        @plsc.parallel_loop(0, CHUNK, step=16, unroll=8)
        def _pass_a(e):
            pk = pk_v[pl.ds(e, 16)]
            pers = jnp.bitwise_and(pk, 1023)
            f = lax.shift_right_logical(pk, 10) - 1
            f0_v[b][pl.ds(e, 16)] = f
            f1_v[b][pl.ds(e, 16)] = f + 1
            item = lax.shift_right_logical(pk, 20)
            aa_v[b][pl.ds(e, 16)] = plsc.load_gather(a_v, [item])
            tt_v[b][pl.ds(e, 16)] = plsc.load_gather(t_v, [pers])
        pltpu.async_copy(tbl_s.at[f0_v[b]], bl_v[b], sem[b])
        pltpu.async_copy(tbl_s.at[f1_v[b]], bu_v[b], sem[b])

    def drain(j, b):
        base = (start_ch + j) * CHUNK
        pltpu.make_async_copy(tbl_s.at[f0_v[b]], bl_v[b], sem[b]).wait()
        pltpu.make_async_copy(tbl_s.at[f1_v[b]], bu_v[b], sem[b]).wait()
        @plsc.parallel_loop(0, CHUNK, step=16, unroll=8)
        def _pass_b(e):
            bl = bl_v[b][pl.ds(e, 16)]
            bu = bu_v[b][pl.ds(e, 16)]
            aa = aa_v[b][pl.ds(e, 16)]
            tt = tt_v[b][pl.ds(e, 16)]
            x1 = jnp.clip(aa * (tt - bl), -85.0, 85.0)
            x2 = jnp.clip(aa * (tt - bu), -85.0, 85.0)
            s1 = 1.0 / (1.0 + jnp.exp(-x1))
            s2 = 1.0 / (1.0 + jnp.exp(-x2))
            out_v[pl.ds(e, 16)] = cvec - _log_poly(s1 - s2)
        pltpu.sync_copy(out_v, out_hbm.at[pl.ds(base, CHUNK)])

    fire(0, 0)

    def pair_body(jj, carry):
        for par in range(2):
            j = 2 * jj + par
            jn = j + 1

            @pl.when(jn < n_ch)
            def _f():
                fire(jn, (par + 1) % 2)

            @pl.when(j < n_ch)
            def _d():
                drain(j, par)
        return carry

    lax.fori_loop(0, (CH_LO + 2) // 2, pair_body, 0)


@functools.partial(jax.jit, static_argnums=())
def _sc_call(pk, tbl, asp, t, cvec):
    mesh = plsc.VectorSubcoreMesh(core_axis_name="c", subcore_axis_name="s")
    return pl.kernel(
        _sc_body,
        out_type=jax.ShapeDtypeStruct((B_TOT,), jnp.float32),
        mesh=mesh,
        scratch_types=[
            pltpu.VMEM((CHUNK,), jnp.int32),
            [pltpu.VMEM((CHUNK,), jnp.int32)] * 2,
            [pltpu.VMEM((CHUNK,), jnp.int32)] * 2,
            [pltpu.VMEM((CHUNK,), jnp.float32)] * 2,
            [pltpu.VMEM((CHUNK,), jnp.float32)] * 2,
            [pltpu.VMEM((CHUNK,), jnp.float32)] * 2,
            [pltpu.VMEM((CHUNK,), jnp.float32)] * 2,
            pltpu.VMEM((CHUNK,), jnp.float32),
            pltpu.VMEM((N_IT,), jnp.float32),
            pltpu.VMEM((N_PE,), jnp.float32),
            pltpu.VMEM((16,), jnp.float32),
            pltpu.VMEM_SHARED((N_IT * W,), jnp.float32),
            [pltpu.SemaphoreType.DMA] * 2,
        ],
        compiler_params=pltpu.CompilerParams(needs_layout_passes=False),
    )(pk, tbl, asp, t, cvec)


def kernel(indices, a_, b_base_, b_diff_, t):
    b = indices.shape[0]
    b_diff_pad = jnp.pad(b_diff_, ((0, 0), (1, W - 1 - b_diff_.shape[1])))
    fmat, asp, cvec = _prep(a_, b_base_, b_diff_pad, t)
    tbl = fmat.reshape(N_IT * W)
    del b
    wpk = jnp.array([1 << 20, 1, 1 << 10], dtype=jnp.int32)
    pk = jnp.sum(indices * wpk, axis=1)
    return _sc_call(pk, tbl, asp, t, cvec)
